# Initial kernel scaffold; baseline (speedup 1.0000x reference)
import jax
import jax.numpy as jnp
from jax.experimental import pallas as pl


def kernel(x, edge_index, edge_attr, Wq, bq, Wk, bk, Wv, bv, Wskip, bskip, W1, b1, gamma, beta):
    raise NotImplementedError("write your pallas kernel here")


# speedup vs baseline: 23.0499x; 23.0499x over previous
import functools

import jax
import jax.numpy as jnp
from jax import lax
from jax.experimental import pallas as pl
from jax.experimental.pallas import tpu as pltpu
from jax.experimental.pallas import tpu_sc as plsc

N = 10000
E = 320000
D = 128
H = 5
C = 128
HC = H * C
NCORE = 2
NSUB = 16
NW = NCORE * NSUB
EPW = E // NW
BLK = 20
NB = EPW // BLK
NPAIR = NB // 2
HP = 16
SRW = 640
NSEG = 2
NBSEG = NB // NSEG

_mesh = plsc.VectorSubcoreMesh(
    core_axis_name="c", subcore_axis_name="s", num_cores=NCORE, num_subcores=NSUB
)


def _proj_body(x_ref, w_ref, b_ref, q_ref, k_ref, v_ref, sk_ref):
    y = jnp.dot(x_ref[...], w_ref[...], preferred_element_type=jnp.float32)
    y = y + b_ref[...]
    scale = jnp.float32(1.0) / jnp.sqrt(jnp.float32(C))
    q_ref[...] = y[:, :HC] * scale
    k_ref[...] = y[:, HC:2 * HC]
    v_ref[...] = y[:, 2 * HC:3 * HC]
    sk_ref[...] = y[:, 3 * HC:]


def _project(x, wcat, bcat):
    rb = 400
    grid = N // rb
    return pl.pallas_call(
        _proj_body,
        grid=(grid,),
        in_specs=[
            pl.BlockSpec((rb, D), lambda i: (i, 0)),
            pl.BlockSpec((D, 3 * HC + D), lambda i: (0, 0)),
            pl.BlockSpec((1, 3 * HC + D), lambda i: (0, 0)),
        ],
        out_specs=[
            pl.BlockSpec((rb, HC), lambda i: (i, 0)),
            pl.BlockSpec((rb, HC), lambda i: (i, 0)),
            pl.BlockSpec((rb, HC), lambda i: (i, 0)),
            pl.BlockSpec((rb, D), lambda i: (i, 0)),
        ],
        out_shape=[
            jax.ShapeDtypeStruct((N, HC), jnp.float32),
            jax.ShapeDtypeStruct((N, HC), jnp.float32),
            jax.ShapeDtypeStruct((N, HC), jnp.float32),
            jax.ShapeDtypeStruct((N, D), jnp.float32),
        ],
    )(x, wcat, bcat)


def _sc_pass1(q_hbm, k_hbm, srcm, dstm, ex_out, den_out,
              src_i, dst_i, qbuf, kbuf, exbuf, zbuf, den_sh,
              qsem0, qsem1, ksem0, ksem1, exsem0, exsem1):
    c = lax.axis_index("c")
    s = lax.axis_index("s")
    wid = s * NCORE + c
    qsems = (qsem0, qsem1)
    ksems = (ksem0, ksem1)
    exsems = (exsem0, exsem1)

    pltpu.sync_copy(srcm.at[wid], src_i)
    pltpu.sync_copy(dstm.at[wid], dst_i)

    def gissue(j, slot):
        pltpu.async_copy(q_hbm.at[dst_i.at[j]], qbuf.at[slot], qsems[slot])
        pltpu.async_copy(k_hbm.at[src_i.at[j]], kbuf.at[slot], ksems[slot])

    def gwait(j, slot):
        pltpu.make_async_copy(q_hbm.at[dst_i.at[j]], qbuf.at[slot], qsems[slot]).wait()
        pltpu.make_async_copy(k_hbm.at[src_i.at[j]], kbuf.at[slot], ksems[slot]).wait()

    gissue(0, 0)

    base = lax.min(s * SRW, N - SRW)

    @pl.loop(0, SRW)
    def _zero(i):
        zbuf[i, :] = jnp.zeros((HP,), jnp.float32)

    pltpu.sync_copy(zbuf, den_sh.at[pl.ds(base, SRW)])
    plsc.subcore_barrier()

    def compute(j, slot):
        qb = qbuf.at[slot]
        kb = kbuf.at[slot]
        eb = exbuf.at[slot]

        @pl.loop(0, BLK)
        def _edge(e):
            iot = lax.iota(jnp.int32, 16)
            row = jnp.zeros((16,), jnp.float32)
            for h in range(H):
                acc = None
                base = h * C
                for t in range(C // 16):
                    a = qb[e, pl.ds(base + t * 16, 16)]
                    b = kb[e, pl.ds(base + t * 16, 16)]
                    acc = a * b if acc is None else acc + a * b
                row = jnp.where(iot == h, jnp.sum(acc), row)
            eb[e, :] = jnp.exp(row)

    def fire(j, slot):
        pltpu.async_copy(exbuf.at[slot], ex_out.at[wid, j], exsems[slot])
        pltpu.sync_copy(exbuf.at[slot], den_sh.at[dst_i.at[j]], add=True)

    def drain(j, slot):
        pltpu.make_async_copy(exbuf.at[slot], ex_out.at[wid, j], exsems[slot]).wait()

    @pl.loop(0, NPAIR)
    def _pair(i):
        a = 2 * i
        gissue(a + 1, 1)
        gwait(a, 0)

        @pl.when(i > 0)
        def _():
            drain(a - 2, 0)

        compute(a, 0)
        fire(a, 0)

        @pl.when(i + 1 < NPAIR)
        def _():
            gissue(a + 2, 0)

        gwait(a + 1, 1)

        @pl.when(i > 0)
        def _():
            drain(a - 1, 1)

        compute(a + 1, 1)
        fire(a + 1, 1)

    drain(NB - 2, 0)
    drain(NB - 1, 1)
    plsc.subcore_barrier()
    pltpu.sync_copy(den_sh.at[pl.ds(base, SRW)],
                    den_out.at[c].at[pl.ds(base, SRW)])


def _pass1(q, k, srcm, dstm):
    f = pl.kernel(
        _sc_pass1,
        out_type=[
            jax.ShapeDtypeStruct((NW, NB, BLK, HP), jnp.float32),
            jax.ShapeDtypeStruct((NCORE, N, HP), jnp.float32),
        ],
        mesh=_mesh,
        compiler_params=pltpu.CompilerParams(needs_layout_passes=False, use_tc_tiling_on_sc=False),
        scratch_types=[
            pltpu.VMEM((NB, BLK), jnp.int32),
            pltpu.VMEM((NB, BLK), jnp.int32),
            pltpu.VMEM((2, BLK, HC), jnp.float32),
            pltpu.VMEM((2, BLK, HC), jnp.float32),
            pltpu.VMEM((2, BLK, HP), jnp.float32),
            pltpu.VMEM((SRW, HP), jnp.float32),
            pltpu.VMEM_SHARED((N, HP), jnp.float32),
            pltpu.SemaphoreType.DMA,
            pltpu.SemaphoreType.DMA,
            pltpu.SemaphoreType.DMA,
            pltpu.SemaphoreType.DMA,
            pltpu.SemaphoreType.DMA,
            pltpu.SemaphoreType.DMA,
        ],
    )
    return f(q, k, srcm, dstm)


def _inv_body(d0_ref, d1_ref, o_ref):
    d = d0_ref[0] + d1_ref[0]
    o_ref[...] = (jnp.float32(1.0) / jnp.float32(H)) / (d + jnp.float32(1e-16))


def _invdenom(den):
    rows = N * HP // 128
    d2 = den.reshape(NCORE, rows, 128)
    out = pl.pallas_call(
        _inv_body,
        grid=(1,),
        in_specs=[
            pl.BlockSpec((1, rows, 128), lambda i: (0, 0, 0)),
            pl.BlockSpec((1, rows, 128), lambda i: (1, 0, 0)),
        ],
        out_specs=pl.BlockSpec((rows, 128), lambda i: (0, 0)),
        out_shape=jax.ShapeDtypeStruct((rows, 128), jnp.float32),
    )(d2, d2)
    return out.reshape(N, HP)


def _sc_pass2(v_hbm, srcm, dstm, ex_hbm, inv_hbm, acc_out,
              src_i, dst_i, vbuf, exb, ib, mb, acc_sh,
              vsem0, vsem1, esem0, esem1, isem0, isem1):
    c = lax.axis_index("c")
    s = lax.axis_index("s")
    wid = s * NCORE + c
    vsems = (vsem0, vsem1)
    esems = (esem0, esem1)
    isems = (isem0, isem1)

    base = lax.min(s * SRW, N - SRW)

    @pl.loop(0, BLK)
    def _zero(i):
        for t in range(D // 16):
            mb[i, pl.ds(t * 16, 16)] = jnp.zeros((16,), jnp.float32)

    @pl.loop(0, SRW // BLK)
    def _zcp(i):
        pltpu.sync_copy(mb, acc_sh.at[pl.ds(base + i * BLK, BLK)])

    plsc.subcore_barrier()

    def gissue(jg, j, slot):
        pltpu.async_copy(v_hbm.at[src_i.at[j]], vbuf.at[slot], vsems[slot])
        pltpu.async_copy(ex_hbm.at[wid, jg], exb.at[slot], esems[slot])
        pltpu.async_copy(inv_hbm.at[dst_i.at[j]], ib.at[slot], isems[slot])

    def gwait(jg, j, slot):
        pltpu.make_async_copy(v_hbm.at[src_i.at[j]], vbuf.at[slot], vsems[slot]).wait()
        pltpu.make_async_copy(ex_hbm.at[wid, jg], exb.at[slot], esems[slot]).wait()
        pltpu.make_async_copy(inv_hbm.at[dst_i.at[j]], ib.at[slot], isems[slot]).wait()

    def compute(j, slot):
        vb = vbuf.at[slot]
        eb = exb.at[slot]
        nb = ib.at[slot]

        @pl.loop(0, BLK)
        def _edge(e):
            ar = eb[e, :] * nb[e, :]
            al = [ar[h] for h in range(H)]
            for t in range(C // 16):
                acc = None
                for h in range(H):
                    vv = vb[e, pl.ds(h * C + t * 16, 16)]
                    term = al[h] * vv
                    acc = term if acc is None else acc + term
                mb[e, pl.ds(t * 16, 16)] = acc

        pltpu.sync_copy(mb, acc_sh.at[dst_i.at[j]], add=True)

    for hh in range(NSEG):
        pltpu.sync_copy(srcm.at[wid].at[pl.ds(hh * NBSEG, NBSEG)], src_i)
        pltpu.sync_copy(dstm.at[wid].at[pl.ds(hh * NBSEG, NBSEG)], dst_i)
        gissue(hh * NBSEG, 0, 0)

        @pl.loop(0, NBSEG // 2)
        def _pair(i):
            a = 2 * i
            ag = hh * NBSEG + a
            gissue(ag + 1, a + 1, 1)
            gwait(ag, a, 0)
            compute(a, 0)

            @pl.when(i + 1 < NBSEG // 2)
            def _():
                gissue(ag + 2, a + 2, 0)

            gwait(ag + 1, a + 1, 1)
            compute(a + 1, 1)

    plsc.subcore_barrier()
    pltpu.sync_copy(acc_sh.at[pl.ds(base, SRW)],
                    acc_out.at[c].at[pl.ds(base, SRW)])


def _pass2(v, srcm, dstm, ex, inv):
    f = pl.kernel(
        _sc_pass2,
        out_type=jax.ShapeDtypeStruct((NCORE, N, D), jnp.float32),
        mesh=_mesh,
        compiler_params=pltpu.CompilerParams(needs_layout_passes=False, use_tc_tiling_on_sc=False),
        scratch_types=[
            pltpu.VMEM((NBSEG, BLK), jnp.int32),
            pltpu.VMEM((NBSEG, BLK), jnp.int32),
            pltpu.VMEM((2, BLK, HC), jnp.float32),
            pltpu.VMEM((2, BLK, HP), jnp.float32),
            pltpu.VMEM((2, BLK, HP), jnp.float32),
            pltpu.VMEM((BLK, D), jnp.float32),
            pltpu.VMEM_SHARED((N, D), jnp.float32),
            pltpu.SemaphoreType.DMA,
            pltpu.SemaphoreType.DMA,
            pltpu.SemaphoreType.DMA,
            pltpu.SemaphoreType.DMA,
            pltpu.SemaphoreType.DMA,
            pltpu.SemaphoreType.DMA,
        ],
    )
    return f(v, srcm, dstm, ex, inv)


def _mlp_body(a0_ref, a1_ref, sk_ref, w_ref, b_ref, h_ref, s_ref, q_ref):
    z = a0_ref[0] + a1_ref[0] + sk_ref[...]
    hh = jnp.dot(z, w_ref[...], preferred_element_type=jnp.float32) + b_ref[...]
    hh = jnp.maximum(hh, jnp.float32(0.0))
    h_ref[...] = hh
    s_ref[...] = jnp.sum(hh, axis=0, keepdims=True)[None]
    q_ref[...] = jnp.sum(hh * hh, axis=0, keepdims=True)[None]


def _mlp(acc, skip, w1, b1):
    rb = 400
    grid = N // rb
    return pl.pallas_call(
        _mlp_body,
        grid=(grid,),
        in_specs=[
            pl.BlockSpec((1, rb, D), lambda i: (0, i, 0)),
            pl.BlockSpec((1, rb, D), lambda i: (1, i, 0)),
            pl.BlockSpec((rb, D), lambda i: (i, 0)),
            pl.BlockSpec((D, D), lambda i: (0, 0)),
            pl.BlockSpec((1, D), lambda i: (0, 0)),
        ],
        out_specs=[
            pl.BlockSpec((rb, D), lambda i: (i, 0)),
            pl.BlockSpec((1, 1, D), lambda i: (i, 0, 0)),
            pl.BlockSpec((1, 1, D), lambda i: (i, 0, 0)),
        ],
        out_shape=[
            jax.ShapeDtypeStruct((N, D), jnp.float32),
            jax.ShapeDtypeStruct((grid, 1, D), jnp.float32),
            jax.ShapeDtypeStruct((grid, 1, D), jnp.float32),
        ],
    )(acc, acc, skip, w1, b1)


def _bn_body(h_ref, s_ref, q_ref, g_ref, b_ref, o_ref):
    ssum = jnp.sum(s_ref[...], axis=0)
    qsum = jnp.sum(q_ref[...], axis=0)
    mean = ssum / jnp.float32(N)
    var = qsum / jnp.float32(N) - mean * mean
    inv = jnp.float32(1.0) / jnp.sqrt(var + jnp.float32(1e-5))
    o_ref[...] = (h_ref[...] - mean) * inv * g_ref[...] + b_ref[...]


def _bnorm(hmat, sums, sumsq, gamma, beta):
    rb = 400
    grid = N // rb
    return pl.pallas_call(
        _bn_body,
        grid=(grid,),
        in_specs=[
            pl.BlockSpec((rb, D), lambda i: (i, 0)),
            pl.BlockSpec((grid, 1, D), lambda i: (0, 0, 0)),
            pl.BlockSpec((grid, 1, D), lambda i: (0, 0, 0)),
            pl.BlockSpec((1, D), lambda i: (0, 0)),
            pl.BlockSpec((1, D), lambda i: (0, 0)),
        ],
        out_specs=pl.BlockSpec((rb, D), lambda i: (i, 0)),
        out_shape=jax.ShapeDtypeStruct((N, D), jnp.float32),
    )(hmat, sums, sumsq, gamma, beta)


def kernel(x, edge_index, edge_attr, Wq, bq, Wk, bk, Wv, bv, Wskip, bskip,
           W1, b1, gamma, beta):
    del edge_attr
    wcat = jnp.concatenate([Wq, Wk, Wv, Wskip], axis=1)
    bcat = jnp.concatenate([bq, bk, bv, bskip])[None, :]
    q, k, v, skip = _project(x, wcat, bcat)

    srcm = edge_index[0].reshape(NW, NB, BLK)
    dstm = edge_index[1].reshape(NW, NB, BLK)

    ex, den = _pass1(q, k, srcm, dstm)
    inv = _invdenom(den)
    acc = _pass2(v, srcm, dstm, ex, inv)
    hmat, sums, sumsq = _mlp(acc, skip, W1, b1[None, :])
    return _bnorm(hmat, sums, sumsq, gamma[None, :], beta[None, :])


# speedup vs baseline: 24.1835x; 1.0492x over previous
import functools

import jax
import jax.numpy as jnp
from jax import lax
from jax.experimental import pallas as pl
from jax.experimental.pallas import tpu as pltpu
from jax.experimental.pallas import tpu_sc as plsc

N = 10000
E = 320000
D = 128
H = 5
C = 128
HC = H * C
NCORE = 2
NSUB = 16
NW = NCORE * NSUB
EPW = E // NW
BLK = 20
NB = EPW // BLK
NPAIR = NB // 2
HP = 16
SRW = 640
NSEG = 2
NBSEG = NB // NSEG

_mesh = plsc.VectorSubcoreMesh(
    core_axis_name="c", subcore_axis_name="s", num_cores=NCORE, num_subcores=NSUB
)


def _proj_body(x_ref, w_ref, b_ref, q_ref, k_ref, v_ref, sk_ref):
    y = jnp.dot(x_ref[...], w_ref[...], preferred_element_type=jnp.float32)
    y = y + b_ref[...]
    scale = jnp.float32(1.0) / jnp.sqrt(jnp.float32(C))
    q_ref[...] = y[:, :HC] * scale
    k_ref[...] = y[:, HC:2 * HC]
    v_ref[...] = y[:, 2 * HC:3 * HC]
    sk_ref[...] = y[:, 3 * HC:]


def _project(x, wcat, bcat):
    rb = 400
    grid = N // rb
    return pl.pallas_call(
        _proj_body,
        grid=(grid,),
        in_specs=[
            pl.BlockSpec((rb, D), lambda i: (i, 0)),
            pl.BlockSpec((D, 3 * HC + D), lambda i: (0, 0)),
            pl.BlockSpec((1, 3 * HC + D), lambda i: (0, 0)),
        ],
        out_specs=[
            pl.BlockSpec((rb, HC), lambda i: (i, 0)),
            pl.BlockSpec((rb, HC), lambda i: (i, 0)),
            pl.BlockSpec((rb, HC), lambda i: (i, 0)),
            pl.BlockSpec((rb, D), lambda i: (i, 0)),
        ],
        out_shape=[
            jax.ShapeDtypeStruct((N, HC), jnp.float32),
            jax.ShapeDtypeStruct((N, HC), jnp.float32),
            jax.ShapeDtypeStruct((N, HC), jnp.float32),
            jax.ShapeDtypeStruct((N, D), jnp.float32),
        ],
    )(x, wcat, bcat)


def _sc_pass1(q_hbm, k_hbm, srcm, dstm, ex_out, den_out,
              src_i, dst_i, qbuf, kbuf, exbuf, zbuf, den_sh,
              qsem0, qsem1, ksem0, ksem1, exsem0, exsem1, adsem0, adsem1):
    c = lax.axis_index("c")
    s = lax.axis_index("s")
    wid = s * NCORE + c
    qsems = (qsem0, qsem1)
    ksems = (ksem0, ksem1)
    exsems = (exsem0, exsem1)
    adsems = (adsem0, adsem1)

    pltpu.sync_copy(srcm.at[wid], src_i)
    pltpu.sync_copy(dstm.at[wid], dst_i)

    def gissue(j, slot):
        pltpu.async_copy(q_hbm.at[dst_i.at[j]], qbuf.at[slot], qsems[slot])
        pltpu.async_copy(k_hbm.at[src_i.at[j]], kbuf.at[slot], ksems[slot])

    def gwait(j, slot):
        pltpu.make_async_copy(q_hbm.at[dst_i.at[j]], qbuf.at[slot], qsems[slot]).wait()
        pltpu.make_async_copy(k_hbm.at[src_i.at[j]], kbuf.at[slot], ksems[slot]).wait()

    gissue(0, 0)

    base = lax.min(s * SRW, N - SRW)

    @pl.loop(0, SRW)
    def _zero(i):
        zbuf[i, :] = jnp.zeros((HP,), jnp.float32)

    pltpu.sync_copy(zbuf, den_sh.at[pl.ds(base, SRW)])
    plsc.subcore_barrier()

    def compute(j, slot):
        qb = qbuf.at[slot]
        kb = kbuf.at[slot]
        eb = exbuf.at[slot]

        @pl.loop(0, BLK)
        def _edge(e):
            iot = lax.iota(jnp.int32, 16)
            row = jnp.zeros((16,), jnp.float32)
            for h in range(H):
                acc = None
                base = h * C
                for t in range(C // 16):
                    a = qb[e, pl.ds(base + t * 16, 16)]
                    b = kb[e, pl.ds(base + t * 16, 16)]
                    acc = a * b if acc is None else acc + a * b
                row = jnp.where(iot == h, jnp.sum(acc), row)
            eb[e, :] = jnp.exp(row)

    def fire(j, slot):
        pltpu.async_copy(exbuf.at[slot], ex_out.at[wid, j], exsems[slot])
        pltpu.async_copy(exbuf.at[slot], den_sh.at[dst_i.at[j]], adsems[slot],
                         add=True)

    def drain(j, slot):
        pltpu.make_async_copy(exbuf.at[slot], ex_out.at[wid, j], exsems[slot]).wait()
        pltpu.make_async_copy(exbuf.at[slot], den_sh.at[dst_i.at[j]],
                              adsems[slot]).wait()

    @pl.loop(0, NPAIR)
    def _pair(i):
        a = 2 * i
        gissue(a + 1, 1)
        gwait(a, 0)

        @pl.when(i > 0)
        def _():
            drain(a - 2, 0)

        compute(a, 0)
        fire(a, 0)

        @pl.when(i + 1 < NPAIR)
        def _():
            gissue(a + 2, 0)

        gwait(a + 1, 1)

        @pl.when(i > 0)
        def _():
            drain(a - 1, 1)

        compute(a + 1, 1)
        fire(a + 1, 1)

    drain(NB - 2, 0)
    drain(NB - 1, 1)
    plsc.subcore_barrier()
    pltpu.sync_copy(den_sh.at[pl.ds(base, SRW)],
                    den_out.at[c].at[pl.ds(base, SRW)])


def _pass1(q, k, srcm, dstm):
    f = pl.kernel(
        _sc_pass1,
        out_type=[
            jax.ShapeDtypeStruct((NW, NB, BLK, HP), jnp.float32),
            jax.ShapeDtypeStruct((NCORE, N, HP), jnp.float32),
        ],
        mesh=_mesh,
        compiler_params=pltpu.CompilerParams(needs_layout_passes=False, use_tc_tiling_on_sc=False),
        scratch_types=[
            pltpu.VMEM((NB, BLK), jnp.int32),
            pltpu.VMEM((NB, BLK), jnp.int32),
            pltpu.VMEM((2, BLK, HC), jnp.float32),
            pltpu.VMEM((2, BLK, HC), jnp.float32),
            pltpu.VMEM((2, BLK, HP), jnp.float32),
            pltpu.VMEM((SRW, HP), jnp.float32),
            pltpu.VMEM_SHARED((N, HP), jnp.float32),
            pltpu.SemaphoreType.DMA,
            pltpu.SemaphoreType.DMA,
            pltpu.SemaphoreType.DMA,
            pltpu.SemaphoreType.DMA,
            pltpu.SemaphoreType.DMA,
            pltpu.SemaphoreType.DMA,
            pltpu.SemaphoreType.DMA,
            pltpu.SemaphoreType.DMA,
        ],
    )
    return f(q, k, srcm, dstm)


def _inv_body(d0_ref, d1_ref, o_ref):
    d = d0_ref[0] + d1_ref[0]
    o_ref[...] = (jnp.float32(1.0) / jnp.float32(H)) / (d + jnp.float32(1e-16))


def _invdenom(den):
    rows = N * HP // 128
    d2 = den.reshape(NCORE, rows, 128)
    out = pl.pallas_call(
        _inv_body,
        grid=(1,),
        in_specs=[
            pl.BlockSpec((1, rows, 128), lambda i: (0, 0, 0)),
            pl.BlockSpec((1, rows, 128), lambda i: (1, 0, 0)),
        ],
        out_specs=pl.BlockSpec((rows, 128), lambda i: (0, 0)),
        out_shape=jax.ShapeDtypeStruct((rows, 128), jnp.float32),
    )(d2, d2)
    return out.reshape(N, HP)


def _sc_pass2(v_hbm, srcm, dstm, ex_hbm, inv_hbm, acc_out,
              src_i, dst_i, vbuf, exb, ib, mb, acc_sh,
              vsem0, vsem1, esem0, esem1, isem0, isem1, msem0, msem1):
    c = lax.axis_index("c")
    s = lax.axis_index("s")
    wid = s * NCORE + c
    vsems = (vsem0, vsem1)
    esems = (esem0, esem1)
    isems = (isem0, isem1)
    msems = (msem0, msem1)

    base = lax.min(s * SRW, N - SRW)

    @pl.loop(0, BLK)
    def _zero(i):
        for t in range(D // 16):
            mb[0, i, pl.ds(t * 16, 16)] = jnp.zeros((16,), jnp.float32)

    @pl.loop(0, SRW // BLK)
    def _zcp(i):
        pltpu.sync_copy(mb.at[0], acc_sh.at[pl.ds(base + i * BLK, BLK)])

    plsc.subcore_barrier()

    def gissue(jg, j, slot):
        pltpu.async_copy(v_hbm.at[src_i.at[j]], vbuf.at[slot], vsems[slot])
        pltpu.async_copy(ex_hbm.at[wid, jg], exb.at[slot], esems[slot])
        pltpu.async_copy(inv_hbm.at[dst_i.at[j]], ib.at[slot], isems[slot])

    def gwait(jg, j, slot):
        pltpu.make_async_copy(v_hbm.at[src_i.at[j]], vbuf.at[slot], vsems[slot]).wait()
        pltpu.make_async_copy(ex_hbm.at[wid, jg], exb.at[slot], esems[slot]).wait()
        pltpu.make_async_copy(inv_hbm.at[dst_i.at[j]], ib.at[slot], isems[slot]).wait()

    def compute(j, slot):
        vb = vbuf.at[slot]
        eb = exb.at[slot]
        nb = ib.at[slot]
        mbs = mb.at[slot]

        @pl.loop(0, BLK)
        def _edge(e):
            ar = eb[e, :] * nb[e, :]
            al = [ar[h] for h in range(H)]
            for t in range(C // 16):
                acc = None
                for h in range(H):
                    vv = vb[e, pl.ds(h * C + t * 16, 16)]
                    term = al[h] * vv
                    acc = term if acc is None else acc + term
                mbs[e, pl.ds(t * 16, 16)] = acc

        pltpu.async_copy(mbs, acc_sh.at[dst_i.at[j]], msems[slot], add=True)

    def mdrain(j, slot):
        pltpu.make_async_copy(mb.at[slot], acc_sh.at[dst_i.at[j]],
                              msems[slot]).wait()

    for hh in range(NSEG):
        pltpu.sync_copy(srcm.at[wid].at[pl.ds(hh * NBSEG, NBSEG)], src_i)
        pltpu.sync_copy(dstm.at[wid].at[pl.ds(hh * NBSEG, NBSEG)], dst_i)
        gissue(hh * NBSEG, 0, 0)

        @pl.loop(0, NBSEG // 2)
        def _pair(i):
            a = 2 * i
            ag = hh * NBSEG + a
            gissue(ag + 1, a + 1, 1)
            gwait(ag, a, 0)

            @pl.when(i > 0)
            def _():
                mdrain(a - 2, 0)

            compute(a, 0)

            @pl.when(i + 1 < NBSEG // 2)
            def _():
                gissue(ag + 2, a + 2, 0)

            gwait(ag + 1, a + 1, 1)

            @pl.when(i > 0)
            def _():
                mdrain(a - 1, 1)

            compute(a + 1, 1)

        mdrain(NBSEG - 2, 0)
        mdrain(NBSEG - 1, 1)

    plsc.subcore_barrier()
    pltpu.sync_copy(acc_sh.at[pl.ds(base, SRW)],
                    acc_out.at[c].at[pl.ds(base, SRW)])


def _pass2(v, srcm, dstm, ex, inv):
    f = pl.kernel(
        _sc_pass2,
        out_type=jax.ShapeDtypeStruct((NCORE, N, D), jnp.float32),
        mesh=_mesh,
        compiler_params=pltpu.CompilerParams(needs_layout_passes=False, use_tc_tiling_on_sc=False),
        scratch_types=[
            pltpu.VMEM((NBSEG, BLK), jnp.int32),
            pltpu.VMEM((NBSEG, BLK), jnp.int32),
            pltpu.VMEM((2, BLK, HC), jnp.float32),
            pltpu.VMEM((2, BLK, HP), jnp.float32),
            pltpu.VMEM((2, BLK, HP), jnp.float32),
            pltpu.VMEM((2, BLK, D), jnp.float32),
            pltpu.VMEM_SHARED((N, D), jnp.float32),
            pltpu.SemaphoreType.DMA,
            pltpu.SemaphoreType.DMA,
            pltpu.SemaphoreType.DMA,
            pltpu.SemaphoreType.DMA,
            pltpu.SemaphoreType.DMA,
            pltpu.SemaphoreType.DMA,
            pltpu.SemaphoreType.DMA,
            pltpu.SemaphoreType.DMA,
        ],
    )
    return f(v, srcm, dstm, ex, inv)


def _mlp_body(a0_ref, a1_ref, sk_ref, w_ref, b_ref, h_ref, s_ref, q_ref):
    z = a0_ref[0] + a1_ref[0] + sk_ref[...]
    hh = jnp.dot(z, w_ref[...], preferred_element_type=jnp.float32) + b_ref[...]
    hh = jnp.maximum(hh, jnp.float32(0.0))
    h_ref[...] = hh
    s_ref[...] = jnp.sum(hh, axis=0, keepdims=True)[None]
    q_ref[...] = jnp.sum(hh * hh, axis=0, keepdims=True)[None]


def _mlp(acc, skip, w1, b1):
    rb = 400
    grid = N // rb
    return pl.pallas_call(
        _mlp_body,
        grid=(grid,),
        in_specs=[
            pl.BlockSpec((1, rb, D), lambda i: (0, i, 0)),
            pl.BlockSpec((1, rb, D), lambda i: (1, i, 0)),
            pl.BlockSpec((rb, D), lambda i: (i, 0)),
            pl.BlockSpec((D, D), lambda i: (0, 0)),
            pl.BlockSpec((1, D), lambda i: (0, 0)),
        ],
        out_specs=[
            pl.BlockSpec((rb, D), lambda i: (i, 0)),
            pl.BlockSpec((1, 1, D), lambda i: (i, 0, 0)),
            pl.BlockSpec((1, 1, D), lambda i: (i, 0, 0)),
        ],
        out_shape=[
            jax.ShapeDtypeStruct((N, D), jnp.float32),
            jax.ShapeDtypeStruct((grid, 1, D), jnp.float32),
            jax.ShapeDtypeStruct((grid, 1, D), jnp.float32),
        ],
    )(acc, acc, skip, w1, b1)


def _bn_body(h_ref, s_ref, q_ref, g_ref, b_ref, o_ref):
    ssum = jnp.sum(s_ref[...], axis=0)
    qsum = jnp.sum(q_ref[...], axis=0)
    mean = ssum / jnp.float32(N)
    var = qsum / jnp.float32(N) - mean * mean
    inv = jnp.float32(1.0) / jnp.sqrt(var + jnp.float32(1e-5))
    o_ref[...] = (h_ref[...] - mean) * inv * g_ref[...] + b_ref[...]


def _bnorm(hmat, sums, sumsq, gamma, beta):
    rb = 400
    grid = N // rb
    return pl.pallas_call(
        _bn_body,
        grid=(grid,),
        in_specs=[
            pl.BlockSpec((rb, D), lambda i: (i, 0)),
            pl.BlockSpec((grid, 1, D), lambda i: (0, 0, 0)),
            pl.BlockSpec((grid, 1, D), lambda i: (0, 0, 0)),
            pl.BlockSpec((1, D), lambda i: (0, 0)),
            pl.BlockSpec((1, D), lambda i: (0, 0)),
        ],
        out_specs=pl.BlockSpec((rb, D), lambda i: (i, 0)),
        out_shape=jax.ShapeDtypeStruct((N, D), jnp.float32),
    )(hmat, sums, sumsq, gamma, beta)


def kernel(x, edge_index, edge_attr, Wq, bq, Wk, bk, Wv, bv, Wskip, bskip,
           W1, b1, gamma, beta):
    del edge_attr
    wcat = jnp.concatenate([Wq, Wk, Wv, Wskip], axis=1)
    bcat = jnp.concatenate([bq, bk, bv, bskip])[None, :]
    q, k, v, skip = _project(x, wcat, bcat)

    srcm = edge_index[0].reshape(NW, NB, BLK)
    dstm = edge_index[1].reshape(NW, NB, BLK)

    ex, den = _pass1(q, k, srcm, dstm)
    inv = _invdenom(den)
    acc = _pass2(v, srcm, dstm, ex, inv)
    hmat, sums, sumsq = _mlp(acc, skip, W1, b1[None, :])
    return _bnorm(hmat, sums, sumsq, gamma[None, :], beta[None, :])


# speedup vs baseline: 33.4950x; 1.3850x over previous
import functools

import jax
import jax.numpy as jnp
from jax import lax
from jax.experimental import pallas as pl
from jax.experimental.pallas import tpu as pltpu
from jax.experimental.pallas import tpu_sc as plsc

N = 10000
E = 320000
D = 128
H = 5
C = 128
HC = H * C
NCORE = 2
NSUB = 16
NW = NCORE * NSUB
EPW = E // NW
BLK = 20
NB = EPW // BLK
NPAIR = NB // 2
HP = 16
SRW = 640
NSEG = 2
NBSEG = NB // NSEG

_mesh = plsc.VectorSubcoreMesh(
    core_axis_name="c", subcore_axis_name="s", num_cores=NCORE, num_subcores=NSUB
)


def _proj_body(x_ref, w_ref, b_ref, q_ref, k_ref, v_ref, sk_ref):
    y = jnp.dot(x_ref[...], w_ref[...], preferred_element_type=jnp.float32)
    y = y + b_ref[...]
    scale = jnp.float32(1.0) / jnp.sqrt(jnp.float32(C))
    q_ref[...] = (y[:, :HC] * scale).astype(jnp.bfloat16)
    k_ref[...] = y[:, HC:2 * HC].astype(jnp.bfloat16)
    v_ref[...] = y[:, 2 * HC:3 * HC].astype(jnp.bfloat16)
    sk_ref[...] = y[:, 3 * HC:]


def _project(x, wcat, bcat):
    rb = 400
    grid = N // rb
    return pl.pallas_call(
        _proj_body,
        grid=(grid,),
        in_specs=[
            pl.BlockSpec((rb, D), lambda i: (i, 0)),
            pl.BlockSpec((D, 3 * HC + D), lambda i: (0, 0)),
            pl.BlockSpec((1, 3 * HC + D), lambda i: (0, 0)),
        ],
        out_specs=[
            pl.BlockSpec((rb, HC), lambda i: (i, 0)),
            pl.BlockSpec((rb, HC), lambda i: (i, 0)),
            pl.BlockSpec((rb, HC), lambda i: (i, 0)),
            pl.BlockSpec((rb, D), lambda i: (i, 0)),
        ],
        out_shape=[
            jax.ShapeDtypeStruct((N, HC), jnp.bfloat16),
            jax.ShapeDtypeStruct((N, HC), jnp.bfloat16),
            jax.ShapeDtypeStruct((N, HC), jnp.bfloat16),
            jax.ShapeDtypeStruct((N, D), jnp.float32),
        ],
    )(x, wcat, bcat)


def _sc_pass1(q_hbm, k_hbm, srcm, dstm, ex_out, den_out,
              src_i, dst_i, qbuf, kbuf, exbuf, zbuf, den_sh,
              qsem0, qsem1, ksem0, ksem1, exsem0, exsem1, adsem0, adsem1):
    c = lax.axis_index("c")
    s = lax.axis_index("s")
    wid = s * NCORE + c
    qsems = (qsem0, qsem1)
    ksems = (ksem0, ksem1)
    exsems = (exsem0, exsem1)
    adsems = (adsem0, adsem1)

    pltpu.sync_copy(srcm.at[wid], src_i)
    pltpu.sync_copy(dstm.at[wid], dst_i)

    def gissue(j, slot):
        pltpu.async_copy(q_hbm.at[dst_i.at[j]], qbuf.at[slot], qsems[slot])
        pltpu.async_copy(k_hbm.at[src_i.at[j]], kbuf.at[slot], ksems[slot])

    def gwait(j, slot):
        pltpu.make_async_copy(q_hbm.at[dst_i.at[j]], qbuf.at[slot], qsems[slot]).wait()
        pltpu.make_async_copy(k_hbm.at[src_i.at[j]], kbuf.at[slot], ksems[slot]).wait()

    gissue(0, 0)

    base = lax.min(s * SRW, N - SRW)

    @pl.loop(0, SRW)
    def _zero(i):
        zbuf[i, :] = jnp.zeros((HP,), jnp.float32)

    pltpu.sync_copy(zbuf, den_sh.at[pl.ds(base, SRW)])
    plsc.subcore_barrier()

    def compute(j, slot):
        qb = qbuf.at[slot]
        kb = kbuf.at[slot]
        eb = exbuf.at[slot]

        @pl.loop(0, BLK)
        def _edge(e):
            iot = lax.iota(jnp.int32, 16)
            row = jnp.zeros((16,), jnp.float32)
            for h in range(H):
                acc = None
                base = h * C
                for t in range(C // 32):
                    a = qb[e, pl.ds(base + t * 32, 32)]
                    b = kb[e, pl.ds(base + t * 32, 32)]
                    ae, ao = plsc.unpack(a, format=plsc.PackFormat.INTERLEAVED,
                                         preferred_element_type=jnp.float32)
                    be, bo = plsc.unpack(b, format=plsc.PackFormat.INTERLEAVED,
                                         preferred_element_type=jnp.float32)
                    term = ae * be + ao * bo
                    acc = term if acc is None else acc + term
                row = jnp.where(iot == h, jnp.sum(acc), row)
            eb[e, :] = jnp.exp(row)

    def fire(j, slot):
        pltpu.async_copy(exbuf.at[slot], ex_out.at[wid, j], exsems[slot])
        pltpu.async_copy(exbuf.at[slot], den_sh.at[dst_i.at[j]], adsems[slot],
                         add=True)

    def drain(j, slot):
        pltpu.make_async_copy(exbuf.at[slot], ex_out.at[wid, j], exsems[slot]).wait()
        pltpu.make_async_copy(exbuf.at[slot], den_sh.at[dst_i.at[j]],
                              adsems[slot]).wait()

    @pl.loop(0, NPAIR)
    def _pair(i):
        a = 2 * i
        gissue(a + 1, 1)
        gwait(a, 0)

        @pl.when(i > 0)
        def _():
            drain(a - 2, 0)

        compute(a, 0)
        fire(a, 0)

        @pl.when(i + 1 < NPAIR)
        def _():
            gissue(a + 2, 0)

        gwait(a + 1, 1)

        @pl.when(i > 0)
        def _():
            drain(a - 1, 1)

        compute(a + 1, 1)
        fire(a + 1, 1)

    drain(NB - 2, 0)
    drain(NB - 1, 1)
    plsc.subcore_barrier()
    pltpu.sync_copy(den_sh.at[pl.ds(base, SRW)],
                    den_out.at[c].at[pl.ds(base, SRW)])


def _pass1(q, k, srcm, dstm):
    f = pl.kernel(
        _sc_pass1,
        out_type=[
            jax.ShapeDtypeStruct((NW, NB, BLK, HP), jnp.float32),
            jax.ShapeDtypeStruct((NCORE, N, HP), jnp.float32),
        ],
        mesh=_mesh,
        compiler_params=pltpu.CompilerParams(needs_layout_passes=False, use_tc_tiling_on_sc=False),
        scratch_types=[
            pltpu.VMEM((NB, BLK), jnp.int32),
            pltpu.VMEM((NB, BLK), jnp.int32),
            pltpu.VMEM((2, BLK, HC), jnp.bfloat16),
            pltpu.VMEM((2, BLK, HC), jnp.bfloat16),
            pltpu.VMEM((2, BLK, HP), jnp.float32),
            pltpu.VMEM((SRW, HP), jnp.float32),
            pltpu.VMEM_SHARED((N, HP), jnp.float32),
            pltpu.SemaphoreType.DMA,
            pltpu.SemaphoreType.DMA,
            pltpu.SemaphoreType.DMA,
            pltpu.SemaphoreType.DMA,
            pltpu.SemaphoreType.DMA,
            pltpu.SemaphoreType.DMA,
            pltpu.SemaphoreType.DMA,
            pltpu.SemaphoreType.DMA,
        ],
    )
    return f(q, k, srcm, dstm)


def _inv_body(d0_ref, d1_ref, o_ref):
    d = d0_ref[0] + d1_ref[0]
    o_ref[...] = (jnp.float32(1.0) / jnp.float32(H)) / (d + jnp.float32(1e-16))


def _invdenom(den):
    rows = N * HP // 128
    d2 = den.reshape(NCORE, rows, 128)
    out = pl.pallas_call(
        _inv_body,
        grid=(1,),
        in_specs=[
            pl.BlockSpec((1, rows, 128), lambda i: (0, 0, 0)),
            pl.BlockSpec((1, rows, 128), lambda i: (1, 0, 0)),
        ],
        out_specs=pl.BlockSpec((rows, 128), lambda i: (0, 0)),
        out_shape=jax.ShapeDtypeStruct((rows, 128), jnp.float32),
    )(d2, d2)
    return out.reshape(N, HP)


def _sc_pass2(v_hbm, srcm, dstm, ex_hbm, inv_hbm, acc_out,
              src_i, dst_i, vbuf, exb, ib, mb, acc_sh,
              vsem0, vsem1, esem0, esem1, isem0, isem1, msem0, msem1):
    c = lax.axis_index("c")
    s = lax.axis_index("s")
    wid = s * NCORE + c
    vsems = (vsem0, vsem1)
    esems = (esem0, esem1)
    isems = (isem0, isem1)
    msems = (msem0, msem1)

    base = lax.min(s * SRW, N - SRW)

    @pl.loop(0, BLK)
    def _zero(i):
        for t in range(D // 16):
            mb[0, i, pl.ds(t * 16, 16)] = jnp.zeros((16,), jnp.float32)

    @pl.loop(0, SRW // BLK)
    def _zcp(i):
        pltpu.sync_copy(mb.at[0], acc_sh.at[pl.ds(base + i * BLK, BLK)])

    plsc.subcore_barrier()

    def gissue(jg, j, slot):
        pltpu.async_copy(v_hbm.at[src_i.at[j]], vbuf.at[slot], vsems[slot])
        pltpu.async_copy(ex_hbm.at[wid, jg], exb.at[slot], esems[slot])
        pltpu.async_copy(inv_hbm.at[dst_i.at[j]], ib.at[slot], isems[slot])

    def gwait(jg, j, slot):
        pltpu.make_async_copy(v_hbm.at[src_i.at[j]], vbuf.at[slot], vsems[slot]).wait()
        pltpu.make_async_copy(ex_hbm.at[wid, jg], exb.at[slot], esems[slot]).wait()
        pltpu.make_async_copy(inv_hbm.at[dst_i.at[j]], ib.at[slot], isems[slot]).wait()

    def compute(j, slot):
        vb = vbuf.at[slot]
        eb = exb.at[slot]
        nb = ib.at[slot]
        mbs = mb.at[slot]

        @pl.loop(0, BLK)
        def _edge(e):
            ar = eb[e, :] * nb[e, :]
            al = [ar[h] for h in range(H)]
            for t in range(C // 32):
                alo = None
                ahi = None
                for h in range(H):
                    vv = vb[e, pl.ds(h * C + t * 32, 32)]
                    ve, vo = plsc.unpack(vv, format=plsc.PackFormat.INTERLEAVED,
                                         preferred_element_type=jnp.float32)
                    tlo = al[h] * ve
                    thi = al[h] * vo
                    alo = tlo if alo is None else alo + tlo
                    ahi = thi if ahi is None else ahi + thi
                mbs[e, pl.ds(t * 32, 16)] = alo
                mbs[e, pl.ds(t * 32 + 16, 16)] = ahi

        pltpu.async_copy(mbs, acc_sh.at[dst_i.at[j]], msems[slot], add=True)

    def mdrain(j, slot):
        pltpu.make_async_copy(mb.at[slot], acc_sh.at[dst_i.at[j]],
                              msems[slot]).wait()

    for hh in range(NSEG):
        pltpu.sync_copy(srcm.at[wid].at[pl.ds(hh * NBSEG, NBSEG)], src_i)
        pltpu.sync_copy(dstm.at[wid].at[pl.ds(hh * NBSEG, NBSEG)], dst_i)
        gissue(hh * NBSEG, 0, 0)

        @pl.loop(0, NBSEG // 2)
        def _pair(i):
            a = 2 * i
            ag = hh * NBSEG + a
            gissue(ag + 1, a + 1, 1)
            gwait(ag, a, 0)

            @pl.when(i > 0)
            def _():
                mdrain(a - 2, 0)

            compute(a, 0)

            @pl.when(i + 1 < NBSEG // 2)
            def _():
                gissue(ag + 2, a + 2, 0)

            gwait(ag + 1, a + 1, 1)

            @pl.when(i > 0)
            def _():
                mdrain(a - 1, 1)

            compute(a + 1, 1)

        mdrain(NBSEG - 2, 0)
        mdrain(NBSEG - 1, 1)

    plsc.subcore_barrier()
    pltpu.sync_copy(acc_sh.at[pl.ds(base, SRW)],
                    acc_out.at[c].at[pl.ds(base, SRW)])


def _pass2(v, srcm, dstm, ex, inv):
    f = pl.kernel(
        _sc_pass2,
        out_type=jax.ShapeDtypeStruct((NCORE, N, D), jnp.float32),
        mesh=_mesh,
        compiler_params=pltpu.CompilerParams(needs_layout_passes=False, use_tc_tiling_on_sc=False),
        scratch_types=[
            pltpu.VMEM((NBSEG, BLK), jnp.int32),
            pltpu.VMEM((NBSEG, BLK), jnp.int32),
            pltpu.VMEM((2, BLK, HC), jnp.bfloat16),
            pltpu.VMEM((2, BLK, HP), jnp.float32),
            pltpu.VMEM((2, BLK, HP), jnp.float32),
            pltpu.VMEM((2, BLK, D), jnp.float32),
            pltpu.VMEM_SHARED((N, D), jnp.float32),
            pltpu.SemaphoreType.DMA,
            pltpu.SemaphoreType.DMA,
            pltpu.SemaphoreType.DMA,
            pltpu.SemaphoreType.DMA,
            pltpu.SemaphoreType.DMA,
            pltpu.SemaphoreType.DMA,
            pltpu.SemaphoreType.DMA,
            pltpu.SemaphoreType.DMA,
        ],
    )
    return f(v, srcm, dstm, ex, inv)


def _mlp_body(a0_ref, a1_ref, sk_ref, w_ref, b_ref, h_ref, s_ref, q_ref):
    z = a0_ref[0] + a1_ref[0] + sk_ref[...]
    hh = jnp.dot(z, w_ref[...], preferred_element_type=jnp.float32) + b_ref[...]
    hh = jnp.maximum(hh, jnp.float32(0.0))
    h_ref[...] = hh
    s_ref[...] = jnp.sum(hh, axis=0, keepdims=True)[None]
    q_ref[...] = jnp.sum(hh * hh, axis=0, keepdims=True)[None]


def _mlp(acc, skip, w1, b1):
    rb = 400
    grid = N // rb
    return pl.pallas_call(
        _mlp_body,
        grid=(grid,),
        in_specs=[
            pl.BlockSpec((1, rb, D), lambda i: (0, i, 0)),
            pl.BlockSpec((1, rb, D), lambda i: (1, i, 0)),
            pl.BlockSpec((rb, D), lambda i: (i, 0)),
            pl.BlockSpec((D, D), lambda i: (0, 0)),
            pl.BlockSpec((1, D), lambda i: (0, 0)),
        ],
        out_specs=[
            pl.BlockSpec((rb, D), lambda i: (i, 0)),
            pl.BlockSpec((1, 1, D), lambda i: (i, 0, 0)),
            pl.BlockSpec((1, 1, D), lambda i: (i, 0, 0)),
        ],
        out_shape=[
            jax.ShapeDtypeStruct((N, D), jnp.float32),
            jax.ShapeDtypeStruct((grid, 1, D), jnp.float32),
            jax.ShapeDtypeStruct((grid, 1, D), jnp.float32),
        ],
    )(acc, acc, skip, w1, b1)


def _bn_body(h_ref, s_ref, q_ref, g_ref, b_ref, o_ref):
    ssum = jnp.sum(s_ref[...], axis=0)
    qsum = jnp.sum(q_ref[...], axis=0)
    mean = ssum / jnp.float32(N)
    var = qsum / jnp.float32(N) - mean * mean
    inv = jnp.float32(1.0) / jnp.sqrt(var + jnp.float32(1e-5))
    o_ref[...] = (h_ref[...] - mean) * inv * g_ref[...] + b_ref[...]


def _bnorm(hmat, sums, sumsq, gamma, beta):
    rb = 400
    grid = N // rb
    return pl.pallas_call(
        _bn_body,
        grid=(grid,),
        in_specs=[
            pl.BlockSpec((rb, D), lambda i: (i, 0)),
            pl.BlockSpec((grid, 1, D), lambda i: (0, 0, 0)),
            pl.BlockSpec((grid, 1, D), lambda i: (0, 0, 0)),
            pl.BlockSpec((1, D), lambda i: (0, 0)),
            pl.BlockSpec((1, D), lambda i: (0, 0)),
        ],
        out_specs=pl.BlockSpec((rb, D), lambda i: (i, 0)),
        out_shape=jax.ShapeDtypeStruct((N, D), jnp.float32),
    )(hmat, sums, sumsq, gamma, beta)


def kernel(x, edge_index, edge_attr, Wq, bq, Wk, bk, Wv, bv, Wskip, bskip,
           W1, b1, gamma, beta):
    del edge_attr
    import numpy as np
    ls = np.empty((HC,), np.int32)
    for b in range(0, HC, 32):
        for i in range(16):
            ls[b + 2 * i] = b + i
            ls[b + 2 * i + 1] = b + 16 + i
    wcat = jnp.concatenate([Wq, Wk, Wv[:, ls], Wskip], axis=1)
    bcat = jnp.concatenate([bq, bk, bv[ls], bskip])[None, :]
    q, k, v, skip = _project(x, wcat, bcat)

    srcm = edge_index[0].reshape(NW, NB, BLK)
    dstm = edge_index[1].reshape(NW, NB, BLK)

    ex, den = _pass1(q, k, srcm, dstm)
    inv = _invdenom(den)
    acc = _pass2(v, srcm, dstm, ex, inv)
    hmat, sums, sumsq = _mlp(acc, skip, W1, b1[None, :])
    return _bnorm(hmat, sums, sumsq, gamma[None, :], beta[None, :])


# speedup vs baseline: 36.5915x; 1.0924x over previous
import functools

import jax
import jax.numpy as jnp
from jax import lax
from jax.experimental import pallas as pl
from jax.experimental.pallas import tpu as pltpu
from jax.experimental.pallas import tpu_sc as plsc

N = 10000
E = 320000
D = 128
H = 5
C = 128
HC = H * C
NCORE = 2
NSUB = 16
NW = NCORE * NSUB
EPW = E // NW
BLK = 40
NB = EPW // BLK
NPAIR = NB // 2
HP = 16
SRW = 640
NSEG = 5
NBSEG = NB // NSEG

_mesh = plsc.VectorSubcoreMesh(
    core_axis_name="c", subcore_axis_name="s", num_cores=NCORE, num_subcores=NSUB
)


def _proj_body(x_ref, w_ref, b_ref, q_ref, k_ref, v_ref, sk_ref):
    y = jnp.dot(x_ref[...], w_ref[...], preferred_element_type=jnp.float32)
    y = y + b_ref[...]
    scale = jnp.float32(1.0) / jnp.sqrt(jnp.float32(C))
    q_ref[...] = (y[:, :HC] * scale).astype(jnp.bfloat16)
    k_ref[...] = y[:, HC:2 * HC].astype(jnp.bfloat16)
    v_ref[...] = y[:, 2 * HC:3 * HC].astype(jnp.bfloat16)
    sk_ref[...] = y[:, 3 * HC:]


def _project(x, wcat, bcat):
    rb = 400
    grid = N // rb
    return pl.pallas_call(
        _proj_body,
        grid=(grid,),
        in_specs=[
            pl.BlockSpec((rb, D), lambda i: (i, 0)),
            pl.BlockSpec((D, 3 * HC + D), lambda i: (0, 0)),
            pl.BlockSpec((1, 3 * HC + D), lambda i: (0, 0)),
        ],
        out_specs=[
            pl.BlockSpec((rb, HC), lambda i: (i, 0)),
            pl.BlockSpec((rb, HC), lambda i: (i, 0)),
            pl.BlockSpec((rb, HC), lambda i: (i, 0)),
            pl.BlockSpec((rb, D), lambda i: (i, 0)),
        ],
        out_shape=[
            jax.ShapeDtypeStruct((N, HC), jnp.bfloat16),
            jax.ShapeDtypeStruct((N, HC), jnp.bfloat16),
            jax.ShapeDtypeStruct((N, HC), jnp.bfloat16),
            jax.ShapeDtypeStruct((N, D), jnp.float32),
        ],
    )(x, wcat, bcat)


def _sc_pass1(q_hbm, k_hbm, srcm, dstm, ex_out, den_out,
              src_i, dst_i, qbuf, kbuf, exbuf, zbuf, den_sh,
              qsem0, qsem1, ksem0, ksem1, exsem0, exsem1, adsem0, adsem1):
    c = lax.axis_index("c")
    s = lax.axis_index("s")
    wid = s * NCORE + c
    qsems = (qsem0, qsem1)
    ksems = (ksem0, ksem1)
    exsems = (exsem0, exsem1)
    adsems = (adsem0, adsem1)

    pltpu.sync_copy(srcm.at[wid], src_i)
    pltpu.sync_copy(dstm.at[wid], dst_i)

    def gissue(j, slot):
        pltpu.async_copy(q_hbm.at[dst_i.at[j]], qbuf.at[slot], qsems[slot])
        pltpu.async_copy(k_hbm.at[src_i.at[j]], kbuf.at[slot], ksems[slot])

    def gwait(j, slot):
        pltpu.make_async_copy(q_hbm.at[dst_i.at[j]], qbuf.at[slot], qsems[slot]).wait()
        pltpu.make_async_copy(k_hbm.at[src_i.at[j]], kbuf.at[slot], ksems[slot]).wait()

    gissue(0, 0)

    base = lax.min(s * SRW, N - SRW)

    @pl.loop(0, SRW)
    def _zero(i):
        zbuf[i, :] = jnp.zeros((HP,), jnp.float32)

    pltpu.sync_copy(zbuf, den_sh.at[pl.ds(base, SRW)])
    plsc.subcore_barrier()

    def compute(j, slot):
        qb = qbuf.at[slot]
        kb = kbuf.at[slot]
        eb = exbuf.at[slot]

        @pl.loop(0, BLK)
        def _edge(e):
            iot = lax.iota(jnp.int32, 16)
            row = jnp.zeros((16,), jnp.float32)
            for h in range(H):
                acc = None
                base = h * C
                for t in range(C // 32):
                    a = qb[e, pl.ds(base + t * 32, 32)]
                    b = kb[e, pl.ds(base + t * 32, 32)]
                    pe, po = plsc.unpack(a * b,
                                         format=plsc.PackFormat.INTERLEAVED,
                                         preferred_element_type=jnp.float32)
                    term = pe + po
                    acc = term if acc is None else acc + term
                row = jnp.where(iot == h, jnp.sum(acc), row)
            eb[e, :] = jnp.exp(row)

    def fire(j, slot):
        pltpu.async_copy(exbuf.at[slot], ex_out.at[wid, j], exsems[slot])
        pltpu.async_copy(exbuf.at[slot], den_sh.at[dst_i.at[j]], adsems[slot],
                         add=True)

    def drain(j, slot):
        pltpu.make_async_copy(exbuf.at[slot], ex_out.at[wid, j], exsems[slot]).wait()
        pltpu.make_async_copy(exbuf.at[slot], den_sh.at[dst_i.at[j]],
                              adsems[slot]).wait()

    @pl.loop(0, NPAIR)
    def _pair(i):
        a = 2 * i
        gissue(a + 1, 1)
        gwait(a, 0)

        @pl.when(i > 0)
        def _():
            drain(a - 2, 0)

        compute(a, 0)
        fire(a, 0)

        @pl.when(i + 1 < NPAIR)
        def _():
            gissue(a + 2, 0)

        gwait(a + 1, 1)

        @pl.when(i > 0)
        def _():
            drain(a - 1, 1)

        compute(a + 1, 1)
        fire(a + 1, 1)

    drain(NB - 2, 0)
    drain(NB - 1, 1)
    plsc.subcore_barrier()
    pltpu.sync_copy(den_sh.at[pl.ds(base, SRW)],
                    den_out.at[c].at[pl.ds(base, SRW)])


def _pass1(q, k, srcm, dstm):
    f = pl.kernel(
        _sc_pass1,
        out_type=[
            jax.ShapeDtypeStruct((NW, NB, BLK, HP), jnp.float32),
            jax.ShapeDtypeStruct((NCORE, N, HP), jnp.float32),
        ],
        mesh=_mesh,
        compiler_params=pltpu.CompilerParams(needs_layout_passes=False, use_tc_tiling_on_sc=False),
        scratch_types=[
            pltpu.VMEM((NB, BLK), jnp.int32),
            pltpu.VMEM((NB, BLK), jnp.int32),
            pltpu.VMEM((2, BLK, HC), jnp.bfloat16),
            pltpu.VMEM((2, BLK, HC), jnp.bfloat16),
            pltpu.VMEM((2, BLK, HP), jnp.float32),
            pltpu.VMEM((SRW, HP), jnp.float32),
            pltpu.VMEM_SHARED((N, HP), jnp.float32),
            pltpu.SemaphoreType.DMA,
            pltpu.SemaphoreType.DMA,
            pltpu.SemaphoreType.DMA,
            pltpu.SemaphoreType.DMA,
            pltpu.SemaphoreType.DMA,
            pltpu.SemaphoreType.DMA,
            pltpu.SemaphoreType.DMA,
            pltpu.SemaphoreType.DMA,
        ],
    )
    return f(q, k, srcm, dstm)


def _inv_body(d0_ref, d1_ref, o_ref):
    d = d0_ref[0] + d1_ref[0]
    o_ref[...] = (jnp.float32(1.0) / jnp.float32(H)) / (d + jnp.float32(1e-16))


def _invdenom(den):
    rows = N * HP // 128
    d2 = den.reshape(NCORE, rows, 128)
    out = pl.pallas_call(
        _inv_body,
        grid=(1,),
        in_specs=[
            pl.BlockSpec((1, rows, 128), lambda i: (0, 0, 0)),
            pl.BlockSpec((1, rows, 128), lambda i: (1, 0, 0)),
        ],
        out_specs=pl.BlockSpec((rows, 128), lambda i: (0, 0)),
        out_shape=jax.ShapeDtypeStruct((rows, 128), jnp.float32),
    )(d2, d2)
    return out.reshape(N, HP)


def _sc_pass2(v_hbm, srcm, dstm, ex_hbm, inv_hbm, acc_out,
              src_i, dst_i, vbuf, exb, ib, mb, acc_sh,
              vsem0, vsem1, esem0, esem1, isem0, isem1, msem0, msem1):
    c = lax.axis_index("c")
    s = lax.axis_index("s")
    wid = s * NCORE + c
    vsems = (vsem0, vsem1)
    esems = (esem0, esem1)
    isems = (isem0, isem1)
    msems = (msem0, msem1)

    base = lax.min(s * SRW, N - SRW)

    @pl.loop(0, BLK)
    def _zero(i):
        for t in range(D // 16):
            mb[0, i, pl.ds(t * 16, 16)] = jnp.zeros((16,), jnp.float32)

    @pl.loop(0, SRW // BLK)
    def _zcp(i):
        pltpu.sync_copy(mb.at[0], acc_sh.at[pl.ds(base + i * BLK, BLK)])

    plsc.subcore_barrier()

    def gissue(jg, j, slot):
        pltpu.async_copy(v_hbm.at[src_i.at[j]], vbuf.at[slot], vsems[slot])
        pltpu.async_copy(ex_hbm.at[wid, jg], exb.at[slot], esems[slot])
        pltpu.async_copy(inv_hbm.at[dst_i.at[j]], ib.at[slot], isems[slot])

    def gwait(jg, j, slot):
        pltpu.make_async_copy(v_hbm.at[src_i.at[j]], vbuf.at[slot], vsems[slot]).wait()
        pltpu.make_async_copy(ex_hbm.at[wid, jg], exb.at[slot], esems[slot]).wait()
        pltpu.make_async_copy(inv_hbm.at[dst_i.at[j]], ib.at[slot], isems[slot]).wait()

    def compute(j, slot):
        vb = vbuf.at[slot]
        eb = exb.at[slot]
        nb = ib.at[slot]
        mbs = mb.at[slot]

        @pl.loop(0, BLK)
        def _edge(e):
            ar = eb[e, :] * nb[e, :]
            al = [ar[h] for h in range(H)]
            for t in range(C // 32):
                alo = None
                ahi = None
                for h in range(H):
                    vv = vb[e, pl.ds(h * C + t * 32, 32)]
                    ve, vo = plsc.unpack(vv, format=plsc.PackFormat.INTERLEAVED,
                                         preferred_element_type=jnp.float32)
                    tlo = al[h] * ve
                    thi = al[h] * vo
                    alo = tlo if alo is None else alo + tlo
                    ahi = thi if ahi is None else ahi + thi
                mbs[e, pl.ds(t * 32, 16)] = alo
                mbs[e, pl.ds(t * 32 + 16, 16)] = ahi

        pltpu.async_copy(mbs, acc_sh.at[dst_i.at[j]], msems[slot], add=True)

    def mdrain(j, slot):
        pltpu.make_async_copy(mb.at[slot], acc_sh.at[dst_i.at[j]],
                              msems[slot]).wait()

    for hh in range(NSEG):
        pltpu.sync_copy(srcm.at[wid].at[pl.ds(hh * NBSEG, NBSEG)], src_i)
        pltpu.sync_copy(dstm.at[wid].at[pl.ds(hh * NBSEG, NBSEG)], dst_i)
        gissue(hh * NBSEG, 0, 0)

        @pl.loop(0, NBSEG // 2)
        def _pair(i):
            a = 2 * i
            ag = hh * NBSEG + a
            gissue(ag + 1, a + 1, 1)
            gwait(ag, a, 0)

            @pl.when(i > 0)
            def _():
                mdrain(a - 2, 0)

            compute(a, 0)

            @pl.when(i + 1 < NBSEG // 2)
            def _():
                gissue(ag + 2, a + 2, 0)

            gwait(ag + 1, a + 1, 1)

            @pl.when(i > 0)
            def _():
                mdrain(a - 1, 1)

            compute(a + 1, 1)

        mdrain(NBSEG - 2, 0)
        mdrain(NBSEG - 1, 1)

    plsc.subcore_barrier()
    pltpu.sync_copy(acc_sh.at[pl.ds(base, SRW)],
                    acc_out.at[c].at[pl.ds(base, SRW)])


def _pass2(v, srcm, dstm, ex, inv):
    f = pl.kernel(
        _sc_pass2,
        out_type=jax.ShapeDtypeStruct((NCORE, N, D), jnp.float32),
        mesh=_mesh,
        compiler_params=pltpu.CompilerParams(needs_layout_passes=False, use_tc_tiling_on_sc=False),
        scratch_types=[
            pltpu.VMEM((NBSEG, BLK), jnp.int32),
            pltpu.VMEM((NBSEG, BLK), jnp.int32),
            pltpu.VMEM((2, BLK, HC), jnp.bfloat16),
            pltpu.VMEM((2, BLK, HP), jnp.float32),
            pltpu.VMEM((2, BLK, HP), jnp.float32),
            pltpu.VMEM((2, BLK, D), jnp.float32),
            pltpu.VMEM_SHARED((N, D), jnp.float32),
            pltpu.SemaphoreType.DMA,
            pltpu.SemaphoreType.DMA,
            pltpu.SemaphoreType.DMA,
            pltpu.SemaphoreType.DMA,
            pltpu.SemaphoreType.DMA,
            pltpu.SemaphoreType.DMA,
            pltpu.SemaphoreType.DMA,
            pltpu.SemaphoreType.DMA,
        ],
    )
    return f(v, srcm, dstm, ex, inv)


def _mlp_body(a0_ref, a1_ref, sk_ref, w_ref, b_ref, h_ref, s_ref, q_ref):
    z = a0_ref[0] + a1_ref[0] + sk_ref[...]
    hh = jnp.dot(z, w_ref[...], preferred_element_type=jnp.float32) + b_ref[...]
    hh = jnp.maximum(hh, jnp.float32(0.0))
    h_ref[...] = hh
    s_ref[...] = jnp.sum(hh, axis=0, keepdims=True)[None]
    q_ref[...] = jnp.sum(hh * hh, axis=0, keepdims=True)[None]


def _mlp(acc, skip, w1, b1):
    rb = 400
    grid = N // rb
    return pl.pallas_call(
        _mlp_body,
        grid=(grid,),
        in_specs=[
            pl.BlockSpec((1, rb, D), lambda i: (0, i, 0)),
            pl.BlockSpec((1, rb, D), lambda i: (1, i, 0)),
            pl.BlockSpec((rb, D), lambda i: (i, 0)),
            pl.BlockSpec((D, D), lambda i: (0, 0)),
            pl.BlockSpec((1, D), lambda i: (0, 0)),
        ],
        out_specs=[
            pl.BlockSpec((rb, D), lambda i: (i, 0)),
            pl.BlockSpec((1, 1, D), lambda i: (i, 0, 0)),
            pl.BlockSpec((1, 1, D), lambda i: (i, 0, 0)),
        ],
        out_shape=[
            jax.ShapeDtypeStruct((N, D), jnp.float32),
            jax.ShapeDtypeStruct((grid, 1, D), jnp.float32),
            jax.ShapeDtypeStruct((grid, 1, D), jnp.float32),
        ],
    )(acc, acc, skip, w1, b1)


def _bn_body(h_ref, s_ref, q_ref, g_ref, b_ref, o_ref):
    ssum = jnp.sum(s_ref[...], axis=0)
    qsum = jnp.sum(q_ref[...], axis=0)
    mean = ssum / jnp.float32(N)
    var = qsum / jnp.float32(N) - mean * mean
    inv = jnp.float32(1.0) / jnp.sqrt(var + jnp.float32(1e-5))
    o_ref[...] = (h_ref[...] - mean) * inv * g_ref[...] + b_ref[...]


def _bnorm(hmat, sums, sumsq, gamma, beta):
    rb = 400
    grid = N // rb
    return pl.pallas_call(
        _bn_body,
        grid=(grid,),
        in_specs=[
            pl.BlockSpec((rb, D), lambda i: (i, 0)),
            pl.BlockSpec((grid, 1, D), lambda i: (0, 0, 0)),
            pl.BlockSpec((grid, 1, D), lambda i: (0, 0, 0)),
            pl.BlockSpec((1, D), lambda i: (0, 0)),
            pl.BlockSpec((1, D), lambda i: (0, 0)),
        ],
        out_specs=pl.BlockSpec((rb, D), lambda i: (i, 0)),
        out_shape=jax.ShapeDtypeStruct((N, D), jnp.float32),
    )(hmat, sums, sumsq, gamma, beta)


def kernel(x, edge_index, edge_attr, Wq, bq, Wk, bk, Wv, bv, Wskip, bskip,
           W1, b1, gamma, beta):
    del edge_attr
    import numpy as np
    ls = np.empty((HC,), np.int32)
    for b in range(0, HC, 32):
        for i in range(16):
            ls[b + 2 * i] = b + i
            ls[b + 2 * i + 1] = b + 16 + i
    wcat = jnp.concatenate([Wq, Wk, Wv[:, ls], Wskip], axis=1)
    bcat = jnp.concatenate([bq, bk, bv[ls], bskip])[None, :]
    q, k, v, skip = _project(x, wcat, bcat)

    srcm = edge_index[0].reshape(NW, NB, BLK)
    dstm = edge_index[1].reshape(NW, NB, BLK)

    ex, den = _pass1(q, k, srcm, dstm)
    inv = _invdenom(den)
    acc = _pass2(v, srcm, dstm, ex, inv)
    hmat, sums, sumsq = _mlp(acc, skip, W1, b1[None, :])
    return _bnorm(hmat, sums, sumsq, gamma[None, :], beta[None, :])


# speedup vs baseline: 37.0326x; 1.0121x over previous
import functools

import jax
import jax.numpy as jnp
from jax import lax
from jax.experimental import pallas as pl
from jax.experimental.pallas import tpu as pltpu
from jax.experimental.pallas import tpu_sc as plsc

N = 10000
E = 320000
D = 128
H = 5
C = 128
HC = H * C
NCORE = 2
NSUB = 16
NW = NCORE * NSUB
EPW = E // NW
BLK = 40
NB = EPW // BLK
NPAIR = NB // 2
HP = 16
SRW = 640
NSEG = 5
NBSEG = NB // NSEG

_mesh = plsc.VectorSubcoreMesh(
    core_axis_name="c", subcore_axis_name="s", num_cores=NCORE, num_subcores=NSUB
)


def _proj_body(x_ref, w_ref, b_ref, q_ref, k_ref, v_ref, sk_ref):
    y = jnp.dot(x_ref[...], w_ref[...], preferred_element_type=jnp.float32)
    y = y + b_ref[...]
    scale = jnp.float32(1.0) / jnp.sqrt(jnp.float32(C))
    q_ref[...] = (y[:, :HC] * scale).astype(jnp.bfloat16)
    k_ref[...] = y[:, HC:2 * HC].astype(jnp.bfloat16)
    v_ref[...] = y[:, 2 * HC:3 * HC].astype(jnp.bfloat16)
    sk_ref[...] = y[:, 3 * HC:]


def _project(x, wcat, bcat):
    rb = 400
    grid = N // rb
    return pl.pallas_call(
        _proj_body,
        grid=(grid,),
        in_specs=[
            pl.BlockSpec((rb, D), lambda i: (i, 0)),
            pl.BlockSpec((D, 3 * HC + D), lambda i: (0, 0)),
            pl.BlockSpec((1, 3 * HC + D), lambda i: (0, 0)),
        ],
        out_specs=[
            pl.BlockSpec((rb, HC), lambda i: (i, 0)),
            pl.BlockSpec((rb, HC), lambda i: (i, 0)),
            pl.BlockSpec((rb, HC), lambda i: (i, 0)),
            pl.BlockSpec((rb, D), lambda i: (i, 0)),
        ],
        out_shape=[
            jax.ShapeDtypeStruct((N, HC), jnp.bfloat16),
            jax.ShapeDtypeStruct((N, HC), jnp.bfloat16),
            jax.ShapeDtypeStruct((N, HC), jnp.bfloat16),
            jax.ShapeDtypeStruct((N, D), jnp.float32),
        ],
    )(x, wcat, bcat)


def _sc_pass1(q_hbm, k_hbm, srcm, dstm, ex_out, den_out,
              src_i, dst_i, qbuf, kbuf, exbuf, zbuf, den_sh,
              qsem0, qsem1, ksem0, ksem1, exsem0, exsem1, adsem0, adsem1):
    c = lax.axis_index("c")
    s = lax.axis_index("s")
    wid = s * NCORE + c
    qsems = (qsem0, qsem1)
    ksems = (ksem0, ksem1)
    exsems = (exsem0, exsem1)
    adsems = (adsem0, adsem1)

    pltpu.sync_copy(srcm.at[wid], src_i)
    pltpu.sync_copy(dstm.at[wid], dst_i)

    def gissue(j, slot):
        pltpu.async_copy(q_hbm.at[dst_i.at[j]], qbuf.at[slot], qsems[slot])
        pltpu.async_copy(k_hbm.at[src_i.at[j]], kbuf.at[slot], ksems[slot])

    def gwait(j, slot):
        pltpu.make_async_copy(q_hbm.at[dst_i.at[j]], qbuf.at[slot], qsems[slot]).wait()
        pltpu.make_async_copy(k_hbm.at[src_i.at[j]], kbuf.at[slot], ksems[slot]).wait()

    gissue(0, 0)

    base = lax.min(s * SRW, N - SRW)

    @pl.loop(0, SRW)
    def _zero(i):
        zbuf[i, :] = jnp.zeros((HP,), jnp.float32)

    pltpu.sync_copy(zbuf, den_sh.at[pl.ds(base, SRW)])
    plsc.subcore_barrier()

    def compute(j, slot):
        qb = qbuf.at[slot]
        kb = kbuf.at[slot]
        eb = exbuf.at[slot]

        @pl.loop(0, BLK)
        def _edge(e):
            iot = lax.iota(jnp.int32, 16)
            row = jnp.zeros((16,), jnp.float32)
            for h in range(H):
                acc = None
                base = h * C
                for t in range(C // 32):
                    a = qb[e, pl.ds(base + t * 32, 32)]
                    b = kb[e, pl.ds(base + t * 32, 32)]
                    p = a * b
                    acc = p if acc is None else acc + p
                pe, po = plsc.unpack(acc, format=plsc.PackFormat.INTERLEAVED,
                                     preferred_element_type=jnp.float32)
                row = jnp.where(iot == h, jnp.sum(pe + po), row)
            eb[e, :] = jnp.exp(row)

    def fire(j, slot):
        pltpu.async_copy(exbuf.at[slot], ex_out.at[wid, j], exsems[slot])
        pltpu.async_copy(exbuf.at[slot], den_sh.at[dst_i.at[j]], adsems[slot],
                         add=True)

    def drain(j, slot):
        pltpu.make_async_copy(exbuf.at[slot], ex_out.at[wid, j], exsems[slot]).wait()
        pltpu.make_async_copy(exbuf.at[slot], den_sh.at[dst_i.at[j]],
                              adsems[slot]).wait()

    @pl.loop(0, NPAIR)
    def _pair(i):
        a = 2 * i
        gissue(a + 1, 1)
        gwait(a, 0)

        @pl.when(i > 0)
        def _():
            drain(a - 2, 0)

        compute(a, 0)
        fire(a, 0)

        @pl.when(i + 1 < NPAIR)
        def _():
            gissue(a + 2, 0)

        gwait(a + 1, 1)

        @pl.when(i > 0)
        def _():
            drain(a - 1, 1)

        compute(a + 1, 1)
        fire(a + 1, 1)

    drain(NB - 2, 0)
    drain(NB - 1, 1)
    plsc.subcore_barrier()
    pltpu.sync_copy(den_sh.at[pl.ds(base, SRW)],
                    den_out.at[c].at[pl.ds(base, SRW)])


def _pass1(q, k, srcm, dstm):
    f = pl.kernel(
        _sc_pass1,
        out_type=[
            jax.ShapeDtypeStruct((NW, NB, BLK, HP), jnp.float32),
            jax.ShapeDtypeStruct((NCORE, N, HP), jnp.float32),
        ],
        mesh=_mesh,
        compiler_params=pltpu.CompilerParams(needs_layout_passes=False, use_tc_tiling_on_sc=False),
        scratch_types=[
            pltpu.VMEM((NB, BLK), jnp.int32),
            pltpu.VMEM((NB, BLK), jnp.int32),
            pltpu.VMEM((2, BLK, HC), jnp.bfloat16),
            pltpu.VMEM((2, BLK, HC), jnp.bfloat16),
            pltpu.VMEM((2, BLK, HP), jnp.float32),
            pltpu.VMEM((SRW, HP), jnp.float32),
            pltpu.VMEM_SHARED((N, HP), jnp.float32),
            pltpu.SemaphoreType.DMA,
            pltpu.SemaphoreType.DMA,
            pltpu.SemaphoreType.DMA,
            pltpu.SemaphoreType.DMA,
            pltpu.SemaphoreType.DMA,
            pltpu.SemaphoreType.DMA,
            pltpu.SemaphoreType.DMA,
            pltpu.SemaphoreType.DMA,
        ],
    )
    return f(q, k, srcm, dstm)


def _inv_body(d0_ref, d1_ref, o_ref):
    d = d0_ref[0] + d1_ref[0]
    o_ref[...] = (jnp.float32(1.0) / jnp.float32(H)) / (d + jnp.float32(1e-16))


def _invdenom(den):
    rows = N * HP // 128
    d2 = den.reshape(NCORE, rows, 128)
    out = pl.pallas_call(
        _inv_body,
        grid=(1,),
        in_specs=[
            pl.BlockSpec((1, rows, 128), lambda i: (0, 0, 0)),
            pl.BlockSpec((1, rows, 128), lambda i: (1, 0, 0)),
        ],
        out_specs=pl.BlockSpec((rows, 128), lambda i: (0, 0)),
        out_shape=jax.ShapeDtypeStruct((rows, 128), jnp.float32),
    )(d2, d2)
    return out.reshape(N, HP)


def _sc_pass2(v_hbm, srcm, dstm, ex_hbm, inv_hbm, acc_out,
              src_i, dst_i, vbuf, exb, ib, mb, acc_sh,
              vsem0, vsem1, esem0, esem1, isem0, isem1, msem0, msem1):
    c = lax.axis_index("c")
    s = lax.axis_index("s")
    wid = s * NCORE + c
    vsems = (vsem0, vsem1)
    esems = (esem0, esem1)
    isems = (isem0, isem1)
    msems = (msem0, msem1)

    base = lax.min(s * SRW, N - SRW)

    @pl.loop(0, BLK)
    def _zero(i):
        for t in range(D // 16):
            mb[0, i, pl.ds(t * 16, 16)] = jnp.zeros((16,), jnp.float32)

    @pl.loop(0, SRW // BLK)
    def _zcp(i):
        pltpu.sync_copy(mb.at[0], acc_sh.at[pl.ds(base + i * BLK, BLK)])

    plsc.subcore_barrier()

    def gissue(jg, j, slot):
        pltpu.async_copy(v_hbm.at[src_i.at[j]], vbuf.at[slot], vsems[slot])
        pltpu.async_copy(ex_hbm.at[wid, jg], exb.at[slot], esems[slot])
        pltpu.async_copy(inv_hbm.at[dst_i.at[j]], ib.at[slot], isems[slot])

    def gwait(jg, j, slot):
        pltpu.make_async_copy(v_hbm.at[src_i.at[j]], vbuf.at[slot], vsems[slot]).wait()
        pltpu.make_async_copy(ex_hbm.at[wid, jg], exb.at[slot], esems[slot]).wait()
        pltpu.make_async_copy(inv_hbm.at[dst_i.at[j]], ib.at[slot], isems[slot]).wait()

    def compute(j, slot):
        vb = vbuf.at[slot]
        eb = exb.at[slot]
        nb = ib.at[slot]
        mbs = mb.at[slot]

        @pl.loop(0, BLK)
        def _edge(e):
            ar = eb[e, :] * nb[e, :]
            al = [ar[h] for h in range(H)]
            for t in range(C // 32):
                alo = None
                ahi = None
                for h in range(H):
                    vv = vb[e, pl.ds(h * C + t * 32, 32)]
                    ve, vo = plsc.unpack(vv, format=plsc.PackFormat.INTERLEAVED,
                                         preferred_element_type=jnp.float32)
                    tlo = al[h] * ve
                    thi = al[h] * vo
                    alo = tlo if alo is None else alo + tlo
                    ahi = thi if ahi is None else ahi + thi
                mbs[e, pl.ds(t * 32, 16)] = alo
                mbs[e, pl.ds(t * 32 + 16, 16)] = ahi

        pltpu.async_copy(mbs, acc_sh.at[dst_i.at[j]], msems[slot], add=True)

    def mdrain(j, slot):
        pltpu.make_async_copy(mb.at[slot], acc_sh.at[dst_i.at[j]],
                              msems[slot]).wait()

    for hh in range(NSEG):
        pltpu.sync_copy(srcm.at[wid].at[pl.ds(hh * NBSEG, NBSEG)], src_i)
        pltpu.sync_copy(dstm.at[wid].at[pl.ds(hh * NBSEG, NBSEG)], dst_i)
        gissue(hh * NBSEG, 0, 0)

        @pl.loop(0, NBSEG // 2)
        def _pair(i):
            a = 2 * i
            ag = hh * NBSEG + a
            gissue(ag + 1, a + 1, 1)
            gwait(ag, a, 0)

            @pl.when(i > 0)
            def _():
                mdrain(a - 2, 0)

            compute(a, 0)

            @pl.when(i + 1 < NBSEG // 2)
            def _():
                gissue(ag + 2, a + 2, 0)

            gwait(ag + 1, a + 1, 1)

            @pl.when(i > 0)
            def _():
                mdrain(a - 1, 1)

            compute(a + 1, 1)

        mdrain(NBSEG - 2, 0)
        mdrain(NBSEG - 1, 1)

    plsc.subcore_barrier()
    pltpu.sync_copy(acc_sh.at[pl.ds(base, SRW)],
                    acc_out.at[c].at[pl.ds(base, SRW)])


def _pass2(v, srcm, dstm, ex, inv):
    f = pl.kernel(
        _sc_pass2,
        out_type=jax.ShapeDtypeStruct((NCORE, N, D), jnp.float32),
        mesh=_mesh,
        compiler_params=pltpu.CompilerParams(needs_layout_passes=False, use_tc_tiling_on_sc=False),
        scratch_types=[
            pltpu.VMEM((NBSEG, BLK), jnp.int32),
            pltpu.VMEM((NBSEG, BLK), jnp.int32),
            pltpu.VMEM((2, BLK, HC), jnp.bfloat16),
            pltpu.VMEM((2, BLK, HP), jnp.float32),
            pltpu.VMEM((2, BLK, HP), jnp.float32),
            pltpu.VMEM((2, BLK, D), jnp.float32),
            pltpu.VMEM_SHARED((N, D), jnp.float32),
            pltpu.SemaphoreType.DMA,
            pltpu.SemaphoreType.DMA,
            pltpu.SemaphoreType.DMA,
            pltpu.SemaphoreType.DMA,
            pltpu.SemaphoreType.DMA,
            pltpu.SemaphoreType.DMA,
            pltpu.SemaphoreType.DMA,
            pltpu.SemaphoreType.DMA,
        ],
    )
    return f(v, srcm, dstm, ex, inv)


def _mlp_body(a0_ref, a1_ref, sk_ref, w_ref, b_ref, h_ref, s_ref, q_ref):
    z = a0_ref[0] + a1_ref[0] + sk_ref[...]
    hh = jnp.dot(z, w_ref[...], preferred_element_type=jnp.float32) + b_ref[...]
    hh = jnp.maximum(hh, jnp.float32(0.0))
    h_ref[...] = hh
    s_ref[...] = jnp.sum(hh, axis=0, keepdims=True)[None]
    q_ref[...] = jnp.sum(hh * hh, axis=0, keepdims=True)[None]


def _mlp(acc, skip, w1, b1):
    rb = 400
    grid = N // rb
    return pl.pallas_call(
        _mlp_body,
        grid=(grid,),
        in_specs=[
            pl.BlockSpec((1, rb, D), lambda i: (0, i, 0)),
            pl.BlockSpec((1, rb, D), lambda i: (1, i, 0)),
            pl.BlockSpec((rb, D), lambda i: (i, 0)),
            pl.BlockSpec((D, D), lambda i: (0, 0)),
            pl.BlockSpec((1, D), lambda i: (0, 0)),
        ],
        out_specs=[
            pl.BlockSpec((rb, D), lambda i: (i, 0)),
            pl.BlockSpec((1, 1, D), lambda i: (i, 0, 0)),
            pl.BlockSpec((1, 1, D), lambda i: (i, 0, 0)),
        ],
        out_shape=[
            jax.ShapeDtypeStruct((N, D), jnp.float32),
            jax.ShapeDtypeStruct((grid, 1, D), jnp.float32),
            jax.ShapeDtypeStruct((grid, 1, D), jnp.float32),
        ],
    )(acc, acc, skip, w1, b1)


def _bn_body(h_ref, s_ref, q_ref, g_ref, b_ref, o_ref):
    ssum = jnp.sum(s_ref[...], axis=0)
    qsum = jnp.sum(q_ref[...], axis=0)
    mean = ssum / jnp.float32(N)
    var = qsum / jnp.float32(N) - mean * mean
    inv = jnp.float32(1.0) / jnp.sqrt(var + jnp.float32(1e-5))
    o_ref[...] = (h_ref[...] - mean) * inv * g_ref[...] + b_ref[...]


def _bnorm(hmat, sums, sumsq, gamma, beta):
    rb = 400
    grid = N // rb
    return pl.pallas_call(
        _bn_body,
        grid=(grid,),
        in_specs=[
            pl.BlockSpec((rb, D), lambda i: (i, 0)),
            pl.BlockSpec((grid, 1, D), lambda i: (0, 0, 0)),
            pl.BlockSpec((grid, 1, D), lambda i: (0, 0, 0)),
            pl.BlockSpec((1, D), lambda i: (0, 0)),
            pl.BlockSpec((1, D), lambda i: (0, 0)),
        ],
        out_specs=pl.BlockSpec((rb, D), lambda i: (i, 0)),
        out_shape=jax.ShapeDtypeStruct((N, D), jnp.float32),
    )(hmat, sums, sumsq, gamma, beta)


def kernel(x, edge_index, edge_attr, Wq, bq, Wk, bk, Wv, bv, Wskip, bskip,
           W1, b1, gamma, beta):
    del edge_attr
    import numpy as np
    ls = np.empty((HC,), np.int32)
    for b in range(0, HC, 32):
        for i in range(16):
            ls[b + 2 * i] = b + i
            ls[b + 2 * i + 1] = b + 16 + i
    wcat = jnp.concatenate([Wq, Wk, Wv[:, ls], Wskip], axis=1)
    bcat = jnp.concatenate([bq, bk, bv[ls], bskip])[None, :]
    q, k, v, skip = _project(x, wcat, bcat)

    srcm = edge_index[0].reshape(NW, NB, BLK)
    dstm = edge_index[1].reshape(NW, NB, BLK)

    ex, den = _pass1(q, k, srcm, dstm)
    inv = _invdenom(den)
    acc = _pass2(v, srcm, dstm, ex, inv)
    hmat, sums, sumsq = _mlp(acc, skip, W1, b1[None, :])
    return _bnorm(hmat, sums, sumsq, gamma[None, :], beta[None, :])
